# baseline (speedup 1.0000x reference)
import dataclasses
import functools

import jax
import jax.numpy as jnp
from jax import lax
from jax.experimental import pallas as pl
from jax.experimental.pallas import tpu as pltpu
from jax.experimental.pallas import tpu_sc as plsc

N = 10000
E = 320000
H = 128
G = 128
NBLK = 2000
EBLK = 12800

SC_NC = 2
SC_NS = 16
SC_NW = SC_NC * SC_NS
CW = 40
NCHUNK = E // CW
NCHW = NCHUNK // SC_NW
ROWS0 = 624


def _bf16_pack_i32(a, b):
    lo = jax.lax.bitcast_convert_type(
        a.astype(jnp.bfloat16), jnp.uint16).astype(jnp.uint32)
    hi = jax.lax.bitcast_convert_type(
        b.astype(jnp.bfloat16), jnp.uint16).astype(jnp.uint32)
    return jax.lax.bitcast_convert_type(lo | (hi << 16), jnp.int32)


def _proj_emit(y, k_ref, qv_ref, s_ref):
    k_ref[...] = y[:, 0:128]
    qv_ref[...] = _bf16_pack_i32(y[:, 128:256], y[:, 256:384])
    s_ref[...] = y[:, 384:512]


def _proj_body(h_ref, w_ref, b_ref, k_ref, qv_ref, s_ref):
    y = jnp.dot(h_ref[...], w_ref[...], preferred_element_type=jnp.float32)
    _proj_emit(y + b_ref[...], k_ref, qv_ref, s_ref)


_PROJ_OUT_SPECS = [
    pl.BlockSpec((NBLK, H), lambda i: (i, 0)),
    pl.BlockSpec((NBLK, H), lambda i: (i, 0)),
    pl.BlockSpec((NBLK, H), lambda i: (i, 0)),
]
_PROJ_OUT_SHAPE = [
    jax.ShapeDtypeStruct((N, H), jnp.float32),
    jax.ShapeDtypeStruct((N, H), jnp.int32),
    jax.ShapeDtypeStruct((N, H), jnp.float32),
]


def _proj(h, wc, bc):
    grid = (N // NBLK,)
    return pl.pallas_call(
        _proj_body,
        grid=grid,
        in_specs=[
            pl.BlockSpec((NBLK, H), lambda i: (i, 0)),
            pl.BlockSpec((H, 4 * H), lambda i: (0, 0)),
            pl.BlockSpec((1, 4 * H), lambda i: (0, 0)),
        ],
        out_specs=_PROJ_OUT_SPECS,
        out_shape=_PROJ_OUT_SHAPE,
    )(h, wc, bc)


def _proj_fused_body(a0_ref, a1_ref, s_ref, w_ref, b_ref,
                     k_ref, qv_ref, s_out_ref):
    h = jnp.maximum(a0_ref[0] + a1_ref[0] + s_ref[...], 0.0)
    y = jnp.dot(h, w_ref[...], preferred_element_type=jnp.float32)
    _proj_emit(y + b_ref[...], k_ref, qv_ref, s_out_ref)


def _proj_fused(agg2, s_prev, wc, bc):
    grid = (N // NBLK,)
    return pl.pallas_call(
        _proj_fused_body,
        grid=grid,
        in_specs=[
            pl.BlockSpec((1, NBLK, H), lambda i: (0, i, 0)),
            pl.BlockSpec((1, NBLK, H), lambda i: (1, i, 0)),
            pl.BlockSpec((NBLK, H), lambda i: (i, 0)),
            pl.BlockSpec((H, 4 * H), lambda i: (0, 0)),
            pl.BlockSpec((1, 4 * H), lambda i: (0, 0)),
        ],
        out_specs=_PROJ_OUT_SPECS,
        out_shape=_PROJ_OUT_SHAPE,
    )(agg2, agg2, s_prev, wc, bc)


def _edge_body(a_ref, w_ref, b_ref, e_ref):
    y = jax.lax.dot_general(
        a_ref[...], w_ref[...], (((0,), (0,)), ((), ())),
        preferred_element_type=jnp.float32)
    y = y + b_ref[...]
    e_ref[...] = _bf16_pack_i32(y[:, 0:64], y[:, 64:128])


def _edge_mm(edge_attr_t, we, be):
    grid = (E // EBLK,)
    return pl.pallas_call(
        _edge_body,
        grid=grid,
        in_specs=[
            pl.BlockSpec((16, EBLK), lambda i: (0, i)),
            pl.BlockSpec((16, H), lambda i: (0, 0)),
            pl.BlockSpec((1, H), lambda i: (0, 0)),
        ],
        out_specs=pl.BlockSpec((EBLK, H // 2), lambda i: (i, 0)),
        out_shape=jax.ShapeDtypeStruct((E, H // 2), jnp.int32),
    )(edge_attr_t, we, be)


def _combine_body(a0_ref, a1_ref, s_ref, o_ref, *, relu):
    y = a0_ref[0] + a1_ref[0] + s_ref[...]
    if relu:
        y = jnp.maximum(y, 0.0)
    o_ref[...] = y


def _combine(agg2, s, relu):
    grid = (N // NBLK,)
    return pl.pallas_call(
        functools.partial(_combine_body, relu=relu),
        grid=grid,
        in_specs=[
            pl.BlockSpec((1, NBLK, H), lambda i: (0, i, 0)),
            pl.BlockSpec((1, NBLK, H), lambda i: (1, i, 0)),
            pl.BlockSpec((NBLK, H), lambda i: (i, 0)),
        ],
        out_specs=pl.BlockSpec((NBLK, H), lambda i: (i, 0)),
        out_shape=jax.ShapeDtypeStruct((N, H), jnp.float32),
    )(agg2, agg2, s)


def _head_body(a0_ref, a1_ref, s_ref, b_ref, w1_ref, b1_ref, w2_ref, b2_ref,
               o_ref, pool_ref):
    i = pl.program_id(0)

    @pl.when(i == 0)
    def _():
        pool_ref[...] = jnp.zeros_like(pool_ref)

    h = a0_ref[0] + a1_ref[0] + s_ref[...]
    seg = b_ref[0, 0, :]
    gids = jax.lax.broadcasted_iota(jnp.int32, (G, NBLK), 0)
    onehot = (seg[None, :] == gids).astype(jnp.float32)
    pool_ref[...] += jnp.dot(onehot, h, preferred_element_type=jnp.float32)

    @pl.when(i == pl.num_programs(0) - 1)
    def _():
        y = jnp.maximum(
            jnp.dot(pool_ref[...], w1_ref[...],
                    preferred_element_type=jnp.float32) + b1_ref[...], 0.0)
        z = jnp.dot(y, w2_ref[...], preferred_element_type=jnp.float32)
        z = z + b2_ref[...]
        o_ref[...] = jax.nn.sigmoid(z)


def _head(agg2, s, batch3, w1, b1, w2, b2):
    grid = (N // NBLK,)
    return pl.pallas_call(
        _head_body,
        grid=grid,
        in_specs=[
            pl.BlockSpec((1, NBLK, H), lambda i: (0, i, 0)),
            pl.BlockSpec((1, NBLK, H), lambda i: (1, i, 0)),
            pl.BlockSpec((NBLK, H), lambda i: (i, 0)),
            pl.BlockSpec((1, 1, NBLK), lambda i: (i, 0, 0)),
            pl.BlockSpec((H, 64), lambda i: (0, 0)),
            pl.BlockSpec((1, 64), lambda i: (0, 0)),
            pl.BlockSpec((64, 1), lambda i: (0, 0)),
            pl.BlockSpec((1, 1), lambda i: (0, 0)),
        ],
        out_specs=pl.BlockSpec((G, 1), lambda i: (0, 0)),
        out_shape=jax.ShapeDtypeStruct((G, 1), jnp.float32),
        scratch_shapes=[pltpu.VMEM((G, H), jnp.float32)],
    )(agg2, agg2, s, batch3, w1, b1, w2, b2)


def _edge_sc_body(kn_hbm, qv_hbm, e_hbm, src_hbm, dst_hbm, zeros_hbm,
                  out_hbm, acc, dsti, srci, kd, qv, ebuf, mb, isem, gsem):
    c = lax.axis_index("c")
    s = lax.axis_index("s")
    wid = s * SC_NC + c

    base_r = s * ROWS0
    pltpu.sync_copy(zeros_hbm.at[pl.ds(base_r, ROWS0)],
                    acc.at[pl.ds(base_r, ROWS0)])

    @pl.when(s == SC_NS - 1)
    def _():
        pltpu.sync_copy(zeros_hbm.at[pl.ds(SC_NS * ROWS0, N - SC_NS * ROWS0)],
                        acc.at[pl.ds(SC_NS * ROWS0, N - SC_NS * ROWS0)])

    plsc.subcore_barrier()

    def issue_idx(t, b):
        base = (wid + t * SC_NW) * CW
        pltpu.async_copy(dst_hbm.at[pl.ds(base, CW)], dsti.at[b], isem)
        pltpu.async_copy(src_hbm.at[pl.ds(base, CW)], srci.at[b], isem)

    def wait_idx(t, b):
        base = (wid + t * SC_NW) * CW
        pltpu.make_async_copy(dst_hbm.at[pl.ds(base, CW)], dsti.at[b],
                              isem).wait()
        pltpu.make_async_copy(src_hbm.at[pl.ds(base, CW)], srci.at[b],
                              isem).wait()

    def issue_data(t, b):
        base = (wid + t * SC_NW) * CW
        pltpu.async_copy(kn_hbm.at[dsti.at[b]], kd.at[b], gsem)
        pltpu.async_copy(qv_hbm.at[srci.at[b]], qv.at[b], gsem)
        pltpu.async_copy(e_hbm.at[pl.ds(base, CW)], ebuf.at[b], gsem)

    def wait_data(t, b):
        base = (wid + t * SC_NW) * CW
        pltpu.make_async_copy(kn_hbm.at[dsti.at[b]], kd.at[b], gsem).wait()
        pltpu.make_async_copy(qv_hbm.at[srci.at[b]], qv.at[b], gsem).wait()
        pltpu.make_async_copy(e_hbm.at[pl.ds(base, CW)], ebuf.at[b], gsem).wait()

    def body(t, b, guard2):
        if guard2:
            @pl.when(t + 1 < NCHW)
            def _():
                wait_idx(t + 1, 1 - b)
                issue_data(t + 1, 1 - b)
        else:
            wait_idx(t + 1, 1 - b)
            issue_data(t + 1, 1 - b)
        wait_data(t, b)

        @pl.loop(0, CW, unroll=4)
        def _(r):
            for u in range(H // 32):
                sla = pl.ds(u * 16, 16)
                slb = pl.ds(64 + u * 16, 16)
                qa, va = plsc.unpack(
                    plsc.bitcast(qv[b, r, sla], jnp.bfloat16),
                    format=plsc.PackFormat.INTERLEAVED)
                qb2, vb2 = plsc.unpack(
                    plsc.bitcast(qv[b, r, slb], jnp.bfloat16),
                    format=plsc.PackFormat.INTERLEAVED)
                ea, eb2 = plsc.unpack(
                    plsc.bitcast(ebuf[b, r, sla], jnp.bfloat16),
                    format=plsc.PackFormat.INTERLEAVED)
                za = kd[b, r, sla] + qa + ea
                zb = kd[b, r, slb] + qb2 + eb2
                mb[r, sla] = va / (1.0 + jnp.exp(za))
                mb[r, slb] = vb2 / (1.0 + jnp.exp(zb))

        pltpu.sync_copy(mb, acc.at[dsti.at[b]], add=True)

        @pl.when(t + 2 < NCHW)
        def _():
            issue_idx(t + 2, b)

    issue_idx(0, 0)
    issue_idx(1, 1)
    wait_idx(0, 0)
    issue_data(0, 0)

    @pl.loop(0, NCHW // 2)
    def _(i):
        t0 = i * 2
        body(t0, 0, guard2=False)
        body(t0 + 1, 1, guard2=True)

    plsc.subcore_barrier()

    pltpu.sync_copy(acc.at[pl.ds(base_r, ROWS0)],
                    out_hbm.at[c, pl.ds(base_r, ROWS0)])

    @pl.when(s == SC_NS - 1)
    def _():
        pltpu.sync_copy(acc.at[pl.ds(SC_NS * ROWS0, N - SC_NS * ROWS0)],
                        out_hbm.at[c, pl.ds(SC_NS * ROWS0, N - SC_NS * ROWS0)])


def _edge_stage(kn, qvt, en, src, dst, zeros):
    mesh = plsc.VectorSubcoreMesh(core_axis_name="c", subcore_axis_name="s")
    cp = pltpu.CompilerParams()
    if "needs_layout_passes" in pltpu.CompilerParams.__dataclass_fields__:
        cp = dataclasses.replace(cp, needs_layout_passes=False)
    f = pl.kernel(
        _edge_sc_body,
        mesh=mesh,
        compiler_params=cp,
        out_type=jax.ShapeDtypeStruct((SC_NC, N, H), jnp.float32),
        scratch_types=[
            pltpu.VMEM_SHARED((N, H), jnp.float32),
            pltpu.VMEM((2, CW), jnp.int32),
            pltpu.VMEM((2, CW), jnp.int32),
            pltpu.VMEM((2, CW, H), jnp.float32),
            pltpu.VMEM((2, CW, H), jnp.int32),
            pltpu.VMEM((2, CW, H // 2), jnp.int32),
            pltpu.VMEM((CW, H), jnp.float32),
            pltpu.SemaphoreType.DMA,
            pltpu.SemaphoreType.DMA,
        ],
    )
    return f(kn, qvt, en, src, dst, zeros)


def _layer_weights(p, din):
    wk = p['W_key']
    pad = H - din
    if pad:
        z = jnp.zeros((pad, wk.shape[1]), jnp.float32)
        cat = lambda w: jnp.concatenate([w, z], axis=0)
    else:
        cat = lambda w: w
    wc = jnp.concatenate(
        [-cat(p['W_key']), -cat(p['W_query']), cat(p['W_value']),
         cat(p['W_skip'])], axis=1)
    bc = jnp.concatenate(
        [-p['b_key'], -p['b_query'], p['b_value'], p['b_skip']])[None, :]
    return wc, bc


def kernel(x, edge_index, edge_attr, batch, params):
    src = edge_index[0]
    dst = edge_index[1]

    ea_t = edge_attr.T
    e_tabs = [
        _edge_mm(ea_t, -params[c]['W_edge'], -params[c]['b_edge'][None, :])
        for c in ('conv1', 'conv2', 'conv3')
    ]
    zeros = jnp.zeros((N, H), jnp.float32)

    h = jnp.pad(x, ((0, 0), (0, H - x.shape[1])))
    wc, bc = _layer_weights(params['conv1'], x.shape[1])
    kn, qvt, s = _proj(h, wc, bc)
    agg2 = _edge_stage(kn, qvt, e_tabs[0], src, dst, zeros)
    for li in (1, 2):
        wc, bc = _layer_weights(params['conv%d' % (li + 1)], H)
        kn, qvt, s_new = _proj_fused(agg2, s, wc, bc)
        agg2 = _edge_stage(kn, qvt, e_tabs[li], src, dst, zeros)
        s = s_new

    batch3 = batch.reshape(N // NBLK, 1, NBLK)
    return _head(agg2, s, batch3,
                 params['lin1']['W'], params['lin1']['b'][None, :],
                 params['lin2']['W'], params['lin2']['b'][None, :])


# speedup vs baseline: 3.2139x; 3.2139x over previous
import dataclasses

import jax
import jax.numpy as jnp
from jax import lax
from jax.experimental import pallas as pl
from jax.experimental.pallas import tpu as pltpu
from jax.experimental.pallas import tpu_sc as plsc

N = 10000
E = 320000
H = 128
G = 128
NBLK = 2000
EBLK = 12800

SC_NC = 2
SC_NS = 16
SC_NW = SC_NC * SC_NS
CW = 40
NCHUNK = E // CW
NCHW = NCHUNK // SC_NW
ROWS0 = 624


def _bf16_pack_i32(a, b):
    lo = jax.lax.bitcast_convert_type(
        a.astype(jnp.bfloat16), jnp.uint16).astype(jnp.uint32)
    hi = jax.lax.bitcast_convert_type(
        b.astype(jnp.bfloat16), jnp.uint16).astype(jnp.uint32)
    return jax.lax.bitcast_convert_type(lo | (hi << 16), jnp.int32)


def _proj_emit(y, k_ref, qv_ref, s_ref):
    k_ref[...] = y[:, 0:128]
    qv_ref[...] = _bf16_pack_i32(y[:, 128:256], y[:, 256:384])
    s_ref[...] = y[:, 384:512]


def _proj_body(h_ref, w_ref, b_ref, k_ref, qv_ref, s_ref):
    y = jnp.dot(h_ref[...], w_ref[...], preferred_element_type=jnp.float32)
    _proj_emit(y + b_ref[...], k_ref, qv_ref, s_ref)


_PROJ_OUT_SPECS = [
    pl.BlockSpec((NBLK, H), lambda i: (i, 0)),
    pl.BlockSpec((NBLK, H), lambda i: (i, 0)),
    pl.BlockSpec((NBLK, H), lambda i: (i, 0)),
]
_PROJ_OUT_SHAPE = [
    jax.ShapeDtypeStruct((N, H), jnp.float32),
    jax.ShapeDtypeStruct((N, H), jnp.int32),
    jax.ShapeDtypeStruct((N, H), jnp.float32),
]


def _proj(h, wc, bc):
    grid = (N // NBLK,)
    return pl.pallas_call(
        _proj_body,
        grid=grid,
        in_specs=[
            pl.BlockSpec((NBLK, H), lambda i: (i, 0)),
            pl.BlockSpec((H, 4 * H), lambda i: (0, 0)),
            pl.BlockSpec((1, 4 * H), lambda i: (0, 0)),
        ],
        out_specs=_PROJ_OUT_SPECS,
        out_shape=_PROJ_OUT_SHAPE,
    )(h, wc, bc)


def _proj_fused_body(a0_ref, a1_ref, s_ref, w_ref, b_ref,
                     k_ref, qv_ref, s_out_ref):
    h = jnp.maximum(a0_ref[0] + a1_ref[0] + s_ref[...], 0.0)
    y = jnp.dot(h, w_ref[...], preferred_element_type=jnp.float32)
    _proj_emit(y + b_ref[...], k_ref, qv_ref, s_out_ref)


def _proj_fused(agg2, s_prev, wc, bc):
    grid = (N // NBLK,)
    return pl.pallas_call(
        _proj_fused_body,
        grid=grid,
        in_specs=[
            pl.BlockSpec((1, NBLK, H), lambda i: (0, i, 0)),
            pl.BlockSpec((1, NBLK, H), lambda i: (1, i, 0)),
            pl.BlockSpec((NBLK, H), lambda i: (i, 0)),
            pl.BlockSpec((H, 4 * H), lambda i: (0, 0)),
            pl.BlockSpec((1, 4 * H), lambda i: (0, 0)),
        ],
        out_specs=_PROJ_OUT_SPECS,
        out_shape=_PROJ_OUT_SHAPE,
    )(agg2, agg2, s_prev, wc, bc)


def _edge_body(a_ref, w_ref, b_ref, e_ref):
    y = jax.lax.dot_general(
        a_ref[...], w_ref[...], (((0,), (0,)), ((), ())),
        preferred_element_type=jnp.float32)
    y = y + b_ref[...]
    e_ref[...] = _bf16_pack_i32(y[:, 0:64], y[:, 64:128])


def _edge_mm(edge_attr_t, we, be):
    grid = (E // EBLK,)
    return pl.pallas_call(
        _edge_body,
        grid=grid,
        in_specs=[
            pl.BlockSpec((16, EBLK), lambda i: (0, i)),
            pl.BlockSpec((16, H), lambda i: (0, 0)),
            pl.BlockSpec((1, H), lambda i: (0, 0)),
        ],
        out_specs=pl.BlockSpec((EBLK, H // 2), lambda i: (i, 0)),
        out_shape=jax.ShapeDtypeStruct((E, H // 2), jnp.int32),
    )(edge_attr_t, we, be)


def _head_body(a0_ref, a1_ref, s_ref, b_ref, w1_ref, b1_ref, w2_ref, b2_ref,
               o_ref, pool_ref):
    i = pl.program_id(0)

    @pl.when(i == 0)
    def _():
        pool_ref[...] = jnp.zeros_like(pool_ref)

    h = a0_ref[0] + a1_ref[0] + s_ref[...]
    seg = b_ref[0, 0, :]
    gids = jax.lax.broadcasted_iota(jnp.int32, (G, NBLK), 0)
    onehot = (seg[None, :] == gids).astype(jnp.float32)
    pool_ref[...] += jnp.dot(onehot, h, preferred_element_type=jnp.float32)

    @pl.when(i == pl.num_programs(0) - 1)
    def _():
        y = jnp.maximum(
            jnp.dot(pool_ref[...], w1_ref[...],
                    preferred_element_type=jnp.float32) + b1_ref[...], 0.0)
        z = jnp.dot(y, w2_ref[...], preferred_element_type=jnp.float32)
        z = z + b2_ref[...]
        o_ref[...] = jax.nn.sigmoid(z)


def _head(agg2, s, batch3, w1, b1, w2, b2):
    grid = (N // NBLK,)
    return pl.pallas_call(
        _head_body,
        grid=grid,
        in_specs=[
            pl.BlockSpec((1, NBLK, H), lambda i: (0, i, 0)),
            pl.BlockSpec((1, NBLK, H), lambda i: (1, i, 0)),
            pl.BlockSpec((NBLK, H), lambda i: (i, 0)),
            pl.BlockSpec((1, 1, NBLK), lambda i: (i, 0, 0)),
            pl.BlockSpec((H, 64), lambda i: (0, 0)),
            pl.BlockSpec((1, 64), lambda i: (0, 0)),
            pl.BlockSpec((64, 1), lambda i: (0, 0)),
            pl.BlockSpec((1, 1), lambda i: (0, 0)),
        ],
        out_specs=pl.BlockSpec((G, 1), lambda i: (0, 0)),
        out_shape=jax.ShapeDtypeStruct((G, 1), jnp.float32),
        scratch_shapes=[pltpu.VMEM((G, H), jnp.float32)],
    )(agg2, agg2, s, batch3, w1, b1, w2, b2)


def _edge_sc_body(kn_hbm, qv_hbm, e_hbm, src_hbm, dst_hbm, zeros_hbm,
                  out_hbm, acc, dsti, srci, kd, qv, ebuf, mb, isem, gsem):
    c = lax.axis_index("c")
    s = lax.axis_index("s")
    wid = s * SC_NC + c

    base_r = s * ROWS0
    pltpu.sync_copy(zeros_hbm.at[pl.ds(base_r, ROWS0)],
                    acc.at[pl.ds(base_r, ROWS0)])

    @pl.when(s == SC_NS - 1)
    def _():
        pltpu.sync_copy(zeros_hbm.at[pl.ds(SC_NS * ROWS0, N - SC_NS * ROWS0)],
                        acc.at[pl.ds(SC_NS * ROWS0, N - SC_NS * ROWS0)])

    plsc.subcore_barrier()

    def issue_idx(t, b):
        base = (wid + t * SC_NW) * CW
        pltpu.async_copy(dst_hbm.at[pl.ds(base, CW)], dsti.at[b], isem)
        pltpu.async_copy(src_hbm.at[pl.ds(base, CW)], srci.at[b], isem)

    def wait_idx(t, b):
        base = (wid + t * SC_NW) * CW
        pltpu.make_async_copy(dst_hbm.at[pl.ds(base, CW)], dsti.at[b],
                              isem).wait()
        pltpu.make_async_copy(src_hbm.at[pl.ds(base, CW)], srci.at[b],
                              isem).wait()

    def issue_data(t, b):
        base = (wid + t * SC_NW) * CW
        pltpu.async_copy(kn_hbm.at[dsti.at[b]], kd.at[b], gsem)
        pltpu.async_copy(qv_hbm.at[srci.at[b]], qv.at[b], gsem)
        pltpu.async_copy(e_hbm.at[pl.ds(base, CW)], ebuf.at[b], gsem)

    def wait_data(t, b):
        base = (wid + t * SC_NW) * CW
        pltpu.make_async_copy(kn_hbm.at[dsti.at[b]], kd.at[b], gsem).wait()
        pltpu.make_async_copy(qv_hbm.at[srci.at[b]], qv.at[b], gsem).wait()
        pltpu.make_async_copy(e_hbm.at[pl.ds(base, CW)], ebuf.at[b], gsem).wait()

    def body(t, b, guard2):
        if guard2:
            @pl.when(t + 1 < NCHW)
            def _():
                wait_idx(t + 1, 1 - b)
                issue_data(t + 1, 1 - b)
        else:
            wait_idx(t + 1, 1 - b)
            issue_data(t + 1, 1 - b)
        wait_data(t, b)

        @pl.loop(0, CW)
        def _(r):
            for u in range(H // 32):
                sla = pl.ds(u * 16, 16)
                slb = pl.ds(64 + u * 16, 16)
                qa, va = plsc.unpack(
                    plsc.bitcast(qv[b, r, sla], jnp.bfloat16),
                    format=plsc.PackFormat.INTERLEAVED)
                qb2, vb2 = plsc.unpack(
                    plsc.bitcast(qv[b, r, slb], jnp.bfloat16),
                    format=plsc.PackFormat.INTERLEAVED)
                ea, eb2 = plsc.unpack(
                    plsc.bitcast(ebuf[b, r, sla], jnp.bfloat16),
                    format=plsc.PackFormat.INTERLEAVED)
                za = kd[b, r, sla] + qa + ea
                zb = kd[b, r, slb] + qb2 + eb2
                mb[r, sla] = va / (1.0 + jnp.exp(za))
                mb[r, slb] = vb2 / (1.0 + jnp.exp(zb))

        pltpu.sync_copy(mb, acc.at[dsti.at[b]], add=True)

        @pl.when(t + 2 < NCHW)
        def _():
            issue_idx(t + 2, b)

    issue_idx(0, 0)
    issue_idx(1, 1)
    wait_idx(0, 0)
    issue_data(0, 0)

    @pl.loop(0, NCHW // 2)
    def _(i):
        t0 = i * 2
        body(t0, 0, guard2=False)
        body(t0 + 1, 1, guard2=True)

    plsc.subcore_barrier()

    pltpu.sync_copy(acc.at[pl.ds(base_r, ROWS0)],
                    out_hbm.at[c, pl.ds(base_r, ROWS0)])

    @pl.when(s == SC_NS - 1)
    def _():
        pltpu.sync_copy(acc.at[pl.ds(SC_NS * ROWS0, N - SC_NS * ROWS0)],
                        out_hbm.at[c, pl.ds(SC_NS * ROWS0, N - SC_NS * ROWS0)])


def _edge_stage(kn, qvt, en, src, dst, zeros):
    mesh = plsc.VectorSubcoreMesh(core_axis_name="c", subcore_axis_name="s")
    cp = pltpu.CompilerParams()
    if "needs_layout_passes" in pltpu.CompilerParams.__dataclass_fields__:
        cp = dataclasses.replace(cp, needs_layout_passes=False)
    f = pl.kernel(
        _edge_sc_body,
        mesh=mesh,
        compiler_params=cp,
        out_type=jax.ShapeDtypeStruct((SC_NC, N, H), jnp.float32),
        scratch_types=[
            pltpu.VMEM_SHARED((N, H), jnp.float32),
            pltpu.VMEM((2, CW), jnp.int32),
            pltpu.VMEM((2, CW), jnp.int32),
            pltpu.VMEM((2, CW, H), jnp.float32),
            pltpu.VMEM((2, CW, H), jnp.int32),
            pltpu.VMEM((2, CW, H // 2), jnp.int32),
            pltpu.VMEM((CW, H), jnp.float32),
            pltpu.SemaphoreType.DMA,
            pltpu.SemaphoreType.DMA,
        ],
    )
    return f(kn, qvt, en, src, dst, zeros)


def _layer_weights(p, din):
    wk = p['W_key']
    pad = H - din
    if pad:
        z = jnp.zeros((pad, wk.shape[1]), jnp.float32)
        cat = lambda w: jnp.concatenate([w, z], axis=0)
    else:
        cat = lambda w: w
    wc = jnp.concatenate(
        [-cat(p['W_key']), -cat(p['W_query']), cat(p['W_value']),
         cat(p['W_skip'])], axis=1)
    bc = jnp.concatenate(
        [-p['b_key'], -p['b_query'], p['b_value'], p['b_skip']])[None, :]
    return wc, bc


def kernel(x, edge_index, edge_attr, batch, params):
    src = edge_index[0]
    dst = edge_index[1]

    ea_t = edge_attr.T
    e_tabs = [
        _edge_mm(ea_t, -params[c]['W_edge'], -params[c]['b_edge'][None, :])
        for c in ('conv1', 'conv2', 'conv3')
    ]
    zeros = jnp.zeros((N, H), jnp.float32)

    h = jnp.pad(x, ((0, 0), (0, H - x.shape[1])))
    wc, bc = _layer_weights(params['conv1'], x.shape[1])
    kn, qvt, s = _proj(h, wc, bc)
    agg2 = _edge_stage(kn, qvt, e_tabs[0], src, dst, zeros)
    for li in (1, 2):
        wc, bc = _layer_weights(params['conv%d' % (li + 1)], H)
        kn, qvt, s_new = _proj_fused(agg2, s, wc, bc)
        agg2 = _edge_stage(kn, qvt, e_tabs[li], src, dst, zeros)
        s = s_new

    batch3 = batch.reshape(N // NBLK, 1, NBLK)
    return _head(agg2, s, batch3,
                 params['lin1']['W'], params['lin1']['b'][None, :],
                 params['lin2']['W'], params['lin2']['b'][None, :])
